# baseline (speedup 1.0000x reference)
import functools

import jax
import jax.numpy as jnp
from jax.experimental import pallas as pl

B, N, CIN, H, CO = 128, 64, 128, 128, 32
TEMP = 0.5
INV_NM1 = 1.0 / (N - 1)

def _gumbel_noise():
    u = jax.random.uniform(jax.random.key(42), (B, N, N), dtype=jnp.float32)
    return -jnp.log(-jnp.log(u + 1e-08) + 1e-08)


def _graph_kernel(x_ref, g_ref, w1m_ref, b1m_ref, w1v_ref, b1v_ref,
                  wmo_ref, bmo_ref, wvo_ref, bvo_ref, out_ref):
    xb = x_ref[0]
    s = jnp.sum(xb, axis=0, keepdims=True)
    agg = (s - xb) * INV_NM1

    hm = jnp.maximum(
        jnp.dot(agg, w1m_ref[...], preferred_element_type=jnp.float32)
        + b1m_ref[...], 0.0)
    mean = (jnp.dot(hm, wmo_ref[...], preferred_element_type=jnp.float32)
            + bmo_ref[...])

    hv = jnp.maximum(
        jnp.dot(agg, w1v_ref[...], preferred_element_type=jnp.float32)
        + b1v_ref[...], 0.0)
    var = (jnp.dot(hv, wvo_ref[...], preferred_element_type=jnp.float32)
           + bvo_ref[...])

    dm = mean[:, None, :] - mean[None, :, :]
    ss = var[:, None, :] + var[None, :, :]
    expo = -(dm * dm) / (2.0 * (ss * ss) + 1e-08)
    p = jnp.mean(jnp.exp(expo), axis=-1)

    logits = (jnp.log(p + 1e-08) + g_ref[0]) * (1.0 / TEMP)
    m = jnp.max(logits, axis=-1, keepdims=True)
    e = jnp.exp(logits - m)
    soft = e / jnp.sum(e, axis=-1, keepdims=True)
    out_ref[0] = (soft > 0.5).astype(jnp.float32)


@functools.partial(jax.jit, static_argnames=())
def _run(x, gumbel, w1mT, b1m, w1vT, b1v, wmoT, bmo, wvoT, bvo):
    full = lambda shape: pl.BlockSpec(shape, lambda b: (0,) * len(shape))
    return pl.pallas_call(
        _graph_kernel,
        grid=(B,),
        in_specs=[
            pl.BlockSpec((1, N, CIN), lambda b: (b, 0, 0)),
            pl.BlockSpec((1, N, N), lambda b: (b, 0, 0)),
            full((CIN, H)), full((1, H)),
            full((CIN, H)), full((1, H)),
            full((H, CO)), full((1, CO)),
            full((H, CO)), full((1, CO)),
        ],
        out_specs=pl.BlockSpec((1, N, N), lambda b: (b, 0, 0)),
        out_shape=jax.ShapeDtypeStruct((B, N, N), jnp.float32),
    )(x, gumbel, w1mT, b1m, w1vT, b1v, wmoT, bmo, wvoT, bvo)


def kernel(x, W1m, b1m, W1v, b1v, Wmo, bmo, Wvo, bvo):
    x = x.astype(jnp.float32)
    return _run(x, _gumbel_noise(),
                W1m.T, b1m[None, :], W1v.T, b1v[None, :],
                Wmo.T, bmo[None, :], Wvo.T, bvo[None, :])


# speedup vs baseline: 3.3565x; 3.3565x over previous
import jax
import jax.numpy as jnp
from jax.experimental import pallas as pl

B, N, CIN, H, CO = 128, 64, 128, 128, 32
TEMP = 0.5
INV_NM1 = 1.0 / (N - 1)
PAIRS = B // 2


def _gumbel_noise():
    u = jax.random.uniform(jax.random.key(42), (B, N, N), dtype=jnp.float32)
    return -jnp.log(-jnp.log(u + 1e-08) + 1e-08)


def _graph_kernel(xt_ref, g_ref, w1m_ref, b1m_ref, w1v_ref, b1v_ref,
                  wmo_ref, bmo_ref, wvo_ref, bvo_ref, out_ref):
    f32 = jnp.float32
    row_half = jax.lax.broadcasted_iota(jnp.int32, (2 * N, 2 * N), 0) // N
    col_half = jax.lax.broadcasted_iota(jnp.int32, (2 * N, 2 * N), 1) // N
    bones = jnp.where(row_half == col_half, f32(1.0), f32(0.0))

    xt = xt_ref[0]
    s = jnp.dot(xt, bones, preferred_element_type=f32)
    agg = (s - xt) * INV_NM1

    hm = jnp.maximum(
        jnp.dot(w1m_ref[...], agg, preferred_element_type=f32)
        + b1m_ref[...], 0.0)
    mt = (jnp.dot(wmo_ref[...], hm, preferred_element_type=f32)
          + bmo_ref[...])
    hv = jnp.maximum(
        jnp.dot(w1v_ref[...], agg, preferred_element_type=f32)
        + b1v_ref[...], 0.0)
    vt = (jnp.dot(wvo_ref[...], hv, preferred_element_type=f32)
          + bvo_ref[...])

    def row_operand(t):
        t0 = t[:, :N, None]
        t1 = t[:, N:, None]
        return jnp.concatenate(
            [jnp.broadcast_to(t0, (CO, N, N)),
             jnp.broadcast_to(t1, (CO, N, N))], axis=2)

    dm = row_operand(mt) - mt[:, None, :]
    ss = row_operand(vt) + vt[:, None, :]
    expo = -(dm * dm) / (2.0 * (ss * ss) + 1e-08)
    p = jnp.sum(jnp.exp(expo), axis=0) * (1.0 / CO)

    logits = (jnp.log(p + 1e-08) + g_ref[0]) * (1.0 / TEMP)
    e = jnp.exp(logits)
    srow = jnp.dot(e, bones, preferred_element_type=f32)
    o = (e > 0.5 * srow).astype(f32)
    out_ref[0, 0] = o[:, :N]
    out_ref[0, 1] = o[:, N:]


@jax.jit
def _run(xt, g2, w1m, b1m, w1v, b1v, wmo, bmo, wvo, bvo):
    full = lambda shape: pl.BlockSpec(shape, lambda k: (0,) * len(shape))
    return pl.pallas_call(
        _graph_kernel,
        grid=(PAIRS,),
        in_specs=[
            pl.BlockSpec((1, CIN, 2 * N), lambda k: (k, 0, 0)),
            pl.BlockSpec((1, N, 2 * N), lambda k: (k, 0, 0)),
            full((H, CIN)), full((H, 1)),
            full((H, CIN)), full((H, 1)),
            full((CO, H)), full((CO, 1)),
            full((CO, H)), full((CO, 1)),
        ],
        out_specs=pl.BlockSpec((1, 2, N, N), lambda k: (k, 0, 0, 0)),
        out_shape=jax.ShapeDtypeStruct((PAIRS, 2, N, N), jnp.float32),
    )(xt, g2, w1m, b1m, w1v, b1v, wmo, bmo, wvo, bvo)


def kernel(x, W1m, b1m, W1v, b1v, Wmo, bmo, Wvo, bvo):
    x = x.astype(jnp.float32)
    xt = x.reshape(PAIRS, 2, N, CIN).transpose(0, 3, 1, 2).reshape(
        PAIRS, CIN, 2 * N)
    g2 = _gumbel_noise().reshape(PAIRS, 2, N, N).transpose(0, 2, 1, 3).reshape(
        PAIRS, N, 2 * N)
    out = _run(xt, g2, W1m, b1m[:, None], W1v, b1v[:, None],
               Wmo, bmo[:, None], Wvo, bvo[:, None])
    return out.reshape(B, N, N)


# speedup vs baseline: 3.4632x; 1.0318x over previous
import jax
import jax.numpy as jnp
from jax.experimental import pallas as pl

B, N, CIN, H, CO = 128, 64, 128, 128, 32
TEMP = 0.5
INV_NM1 = 1.0 / (N - 1)
PAIRS = B // 2


def _gumbel_noise():
    u = jax.random.uniform(jax.random.key(42), (B, N, N), dtype=jnp.float32)
    return -jnp.log(-jnp.log(u + 1e-08) + 1e-08)


def _halves(t2d, red):
    r0 = red(t2d[:, :N], axis=-1, keepdims=True)
    r1 = red(t2d[:, N:], axis=-1, keepdims=True)
    return jnp.concatenate([jnp.broadcast_to(r0, (N, N)),
                            jnp.broadcast_to(r1, (N, N))], axis=1)


def _graph_kernel(x_ref, g_ref, w1m_ref, b1m_ref, w1v_ref, b1v_ref,
                  wmo_ref, bmo_ref, wvo_ref, bvo_ref, out_ref):
    f32 = jnp.float32
    xb = x_ref[...]
    s = jnp.sum(xb, axis=1, keepdims=True)
    agg = ((s - xb) * INV_NM1).reshape(2 * N, CIN)

    hm = jnp.maximum(
        jnp.dot(agg, w1m_ref[...], preferred_element_type=f32)
        + b1m_ref[...], 0.0)
    mean = (jnp.dot(hm, wmo_ref[...], preferred_element_type=f32)
            + bmo_ref[...])
    hv = jnp.maximum(
        jnp.dot(agg, w1v_ref[...], preferred_element_type=f32)
        + b1v_ref[...], 0.0)
    var = (jnp.dot(hv, wvo_ref[...], preferred_element_type=f32)
           + bvo_ref[...])

    mt = mean.T
    vt = var.T

    def row_operand(t):
        t0 = t[:, :N, None]
        t1 = t[:, N:, None]
        return jnp.concatenate(
            [jnp.broadcast_to(t0, (CO, N, N)),
             jnp.broadcast_to(t1, (CO, N, N))], axis=2)

    dm = row_operand(mt) - mt[:, None, :]
    ss = row_operand(vt) + vt[:, None, :]
    expo = -(dm * dm) / (2.0 * (ss * ss) + 1e-08)
    ex = jnp.exp(expo)
    while ex.shape[0] > 1:
        h = ex.shape[0] // 2
        ex = ex[:h] + ex[h:]
    p = ex[0] * (1.0 / CO)

    logits = (jnp.log(p + 1e-08) + g_ref[0]) * (1.0 / TEMP)
    e = jnp.exp(logits - _halves(logits, jnp.max))
    soft = e / _halves(e, jnp.sum)
    o = (soft > 0.5).astype(f32)
    out_ref[0, 0] = o[:, :N]
    out_ref[0, 1] = o[:, N:]


@jax.jit
def _run(x, g2, w1mT, b1m, w1vT, b1v, wmoT, bmo, wvoT, bvo):
    full = lambda shape: pl.BlockSpec(shape, lambda k: (0,) * len(shape))
    return pl.pallas_call(
        _graph_kernel,
        grid=(PAIRS,),
        in_specs=[
            pl.BlockSpec((2, N, CIN), lambda k: (k, 0, 0)),
            pl.BlockSpec((1, N, 2 * N), lambda k: (k, 0, 0)),
            full((CIN, H)), full((1, H)),
            full((CIN, H)), full((1, H)),
            full((H, CO)), full((1, CO)),
            full((H, CO)), full((1, CO)),
        ],
        out_specs=pl.BlockSpec((1, 2, N, N), lambda k: (k, 0, 0, 0)),
        out_shape=jax.ShapeDtypeStruct((PAIRS, 2, N, N), jnp.float32),
    )(x, g2, w1mT, b1m, w1vT, b1v, wmoT, bmo, wvoT, bvo)


def kernel(x, W1m, b1m, W1v, b1v, Wmo, bmo, Wvo, bvo):
    x = x.astype(jnp.float32)
    g2 = _gumbel_noise().reshape(PAIRS, 2, N, N).transpose(0, 2, 1, 3).reshape(
        PAIRS, N, 2 * N)
    out = _run(x, g2, W1m.T, b1m[None, :], W1v.T, b1v[None, :],
               Wmo.T, bmo[None, :], Wvo.T, bvo[None, :])
    return out.reshape(B, N, N)


# speedup vs baseline: 3.7372x; 1.0791x over previous
import jax
import jax.numpy as jnp
from jax.experimental import pallas as pl

B, N, CIN, H, CO = 128, 64, 128, 128, 32
TEMP = 0.5
INV_NM1 = 1.0 / (N - 1)
SAMP = 4
GRID = B // SAMP


def _gumbel_noise():
    u = jax.random.uniform(jax.random.key(42), (B, N, N), dtype=jnp.float32)
    return -jnp.log(-jnp.log(u + 1e-08) + 1e-08)


def _halves(t2d, red):
    r0 = red(t2d[:, :N], axis=-1, keepdims=True)
    r1 = red(t2d[:, N:], axis=-1, keepdims=True)
    return jnp.concatenate([jnp.broadcast_to(r0, (N, N)),
                            jnp.broadcast_to(r1, (N, N))], axis=1)


def _graph_kernel(x_ref, g_ref, w1m_ref, b1m_ref, w1v_ref, b1v_ref,
                  wmo_ref, bmo_ref, wvo_ref, bvo_ref, out_ref):
    f32 = jnp.float32
    xb = x_ref[...]
    s = jnp.sum(xb, axis=1, keepdims=True)
    agg = ((s - xb) * INV_NM1).reshape(SAMP * N, CIN)

    hm = jnp.maximum(
        jnp.dot(agg, w1m_ref[...], preferred_element_type=f32)
        + b1m_ref[...], 0.0)
    mean = (jnp.dot(hm, wmo_ref[...], preferred_element_type=f32)
            + bmo_ref[...])
    hv = jnp.maximum(
        jnp.dot(agg, w1v_ref[...], preferred_element_type=f32)
        + b1v_ref[...], 0.0)
    var = (jnp.dot(hv, wvo_ref[...], preferred_element_type=f32)
           + bvo_ref[...])

    mt = mean.T
    vt = var.T

    def row_operand(t):
        t0 = t[:, :N, None]
        t1 = t[:, N:, None]
        return jnp.concatenate(
            [jnp.broadcast_to(t0, (CO, N, N)),
             jnp.broadcast_to(t1, (CO, N, N))], axis=2)

    for pair in range(SAMP // 2):
        mp = mt[:, 2 * N * pair:2 * N * (pair + 1)]
        vp = vt[:, 2 * N * pair:2 * N * (pair + 1)]
        dm = row_operand(mp) - mp[:, None, :]
        ss = row_operand(vp) + vp[:, None, :]
        expo = -(dm * dm) / (2.0 * (ss * ss) + 1e-08)
        ex = jnp.exp(expo)
        while ex.shape[0] > 1:
            h = ex.shape[0] // 2
            ex = ex[:h] + ex[h:]
        p = ex[0] * (1.0 / CO)

        logits = (jnp.log(p + 1e-08) + g_ref[pair]) * (1.0 / TEMP)
        e = jnp.exp(logits - _halves(logits, jnp.max))
        soft = e / _halves(e, jnp.sum)
        o = (soft > 0.5).astype(f32)
        out_ref[0, 2 * pair] = o[:, :N]
        out_ref[0, 2 * pair + 1] = o[:, N:]


@jax.jit
def _run(x, g2, w1mT, b1m, w1vT, b1v, wmoT, bmo, wvoT, bvo):
    full = lambda shape: pl.BlockSpec(shape, lambda k: (0,) * len(shape))
    return pl.pallas_call(
        _graph_kernel,
        grid=(GRID,),
        in_specs=[
            pl.BlockSpec((SAMP, N, CIN), lambda k: (k, 0, 0)),
            pl.BlockSpec((SAMP // 2, N, 2 * N), lambda k: (k, 0, 0)),
            full((CIN, H)), full((1, H)),
            full((CIN, H)), full((1, H)),
            full((H, CO)), full((1, CO)),
            full((H, CO)), full((1, CO)),
        ],
        out_specs=pl.BlockSpec((1, SAMP, N, N), lambda k: (k, 0, 0, 0)),
        out_shape=jax.ShapeDtypeStruct((GRID, SAMP, N, N), jnp.float32),
    )(x, g2, w1mT, b1m, w1vT, b1v, wmoT, bmo, wvoT, bvo)


def kernel(x, W1m, b1m, W1v, b1v, Wmo, bmo, Wvo, bvo):
    x = x.astype(jnp.float32)
    g2 = _gumbel_noise().reshape(B // 2, 2, N, N).transpose(0, 2, 1, 3).reshape(
        B // 2, N, 2 * N)
    out = _run(x, g2, W1m.T, b1m[None, :], W1v.T, b1v[None, :],
               Wmo.T, bmo[None, :], Wvo.T, bvo[None, :])
    return out.reshape(B, N, N)
